# baseline (speedup 1.0000x reference)
import jax
import jax.numpy as jnp
from jax import lax
from jax.experimental import pallas as pl
from jax.experimental.pallas import tpu as pltpu
from jax.experimental.pallas import tpu_sc as plsc

N = 10000
E = 320000
F = 128
NG = 20
CUTOFF = 5.0
NMOL = 10

NWORK = 32
EPW = E // NWORK
CH = 80
NCH = EPW // CH
NSUB = 16
NPAD = 10240
RPS = NPAD // NSUB

_mesh = plsc.VectorSubcoreMesh(core_axis_name="c", subcore_axis_name="s",
                               num_cores=2, num_subcores=16)


def _edge_body(src_hbm, dst_hbm, pv_hbm, w_hbm, unit_hbm, zeros_hbm,
               out_hbm,
               idx_s, idx_d, pv_r, w_r, unit_r, msg, acc, sem):
    cid = lax.axis_index("c")
    sid = lax.axis_index("s")
    wid = sid * 2 + cid
    pltpu.sync_copy(zeros_hbm.at[pl.ds(sid * RPS, RPS)],
                    acc.at[pl.ds(sid * RPS, RPS)])
    plsc.subcore_barrier()
    base = wid * EPW

    def chunk(i, carry):
        off = base + i * CH
        pltpu.sync_copy(src_hbm.at[pl.ds(off, CH)], idx_s)
        pltpu.sync_copy(dst_hbm.at[pl.ds(off, CH)], idx_d)
        pltpu.async_copy(pv_hbm.at[idx_d], pv_r, sem).wait()
        pltpu.sync_copy(w_hbm.at[pl.ds(off, CH), :], w_r)
        pltpu.sync_copy(unit_hbm.at[pl.ds(off * 4, CH * 4)], unit_r)

        def edge(e, c2):
            ph = [pv_r[e, pl.ds(16 * t, 16)] for t in range(6)]
            wv = [w_r[e, pl.ds(16 * t, 16)] for t in range(6)]
            vv = [pv_r[e, pl.ds(96 + 16 * t, 16)] for t in range(6)]
            msg[e, pl.ds(0, 16)] = ph[0] * wv[0]
            msg[e, pl.ds(16, 16)] = ph[1] * wv[1]
            avs = (ph[2] * wv[2], ph[3] * wv[3])
            avd = (ph[4] * wv[4], ph[5] * wv[5])
            uv = unit_r[pl.ds(e * 4, 16)]
            for k in range(3):
                uk = jnp.full((16,), uv[k], jnp.float32)
                for h in range(2):
                    msg[e, pl.ds(32 + 32 * k + 16 * h, 16)] = (
                        avs[h] * vv[2 * k + h] + avd[h] * uk)
            return c2

        lax.fori_loop(0, CH, edge, 0)
        pltpu.sync_copy(msg, acc.at[idx_s], add=True)
        return carry

    lax.fori_loop(0, NCH, chunk, 0)
    plsc.subcore_barrier()
    pltpu.sync_copy(acc.at[pl.ds(sid * RPS, RPS)],
                    out_hbm.at[cid, pl.ds(sid * RPS, RPS)])


_edge_call = pl.kernel(
    _edge_body,
    out_type=jax.ShapeDtypeStruct((2, NPAD, 128), jnp.float32),
    mesh=_mesh,
    scratch_types=[
        pltpu.VMEM((CH,), jnp.int32),
        pltpu.VMEM((CH,), jnp.int32),
        pltpu.VMEM((CH, 256), jnp.float32),
        pltpu.VMEM((CH, 96), jnp.float32),
        pltpu.VMEM((CH * 4,), jnp.float32),
        pltpu.VMEM((CH, 128), jnp.float32),
        pltpu.VMEM_SHARED((NPAD, 128), jnp.float32),
        pltpu.SemaphoreType.DMA,
    ],
)


_NB = 1000
_EB = 2000


def _phi_body(s_ref, w1_ref, b1_ref, w2_ref, b2_ref, o_ref):
    h = jnp.dot(s_ref[...], w1_ref[...], preferred_element_type=jnp.float32)
    h = h + b1_ref[...]
    h = h * jax.nn.sigmoid(h)
    o_ref[...] = jnp.dot(h, w2_ref[...],
                         preferred_element_type=jnp.float32) + b2_ref[...]


def _phi_mlp(s, w1, b1, w2p, b2p):
    return pl.pallas_call(
        _phi_body,
        grid=(N // _NB,),
        in_specs=[
            pl.BlockSpec((_NB, 128), lambda i: (i, 0)),
            pl.BlockSpec((128, 128), lambda i: (0, 0)),
            pl.BlockSpec((1, 128), lambda i: (0, 0)),
            pl.BlockSpec((128, 384), lambda i: (0, 0)),
            pl.BlockSpec((1, 384), lambda i: (0, 0)),
        ],
        out_specs=pl.BlockSpec((_NB, 384), lambda i: (i, 0)),
        out_shape=jax.ShapeDtypeStruct((N, 384), jnp.float32),
    )(s, w1, b1[None, :], w2p, b2p[None, :])


def _w_body(erbf_ref, wd_ref, o_ref):
    o_ref[...] = jnp.dot(erbf_ref[...], wd_ref[...],
                         preferred_element_type=jnp.float32)


def _w_matmul(erbf21, wd21):
    return pl.pallas_call(
        _w_body,
        grid=(E // _EB,),
        in_specs=[
            pl.BlockSpec((_EB, 24), lambda i: (i, 0)),
            pl.BlockSpec((24, 384), lambda i: (0, 0)),
        ],
        out_specs=pl.BlockSpec((_EB, 384), lambda i: (i, 0)),
        out_shape=jax.ShapeDtypeStruct((E, 384), jnp.float32),
    )(erbf21, wd21)


def _upd_body(s_ref, v_ref, u_ref, vw_ref, w1s_ref, w1v_ref, b1_ref,
              w2_ref, b2_ref, so_ref, vo_ref):
    v = v_ref[...]
    uv = jnp.dot(v, u_ref[...], preferred_element_type=jnp.float32)
    vv = jnp.dot(v, vw_ref[...], preferred_element_type=jnp.float32)
    vv3 = vv.reshape(_NB, 3, 128)
    uv3 = uv.reshape(_NB, 3, 128)
    vn = jnp.sqrt(jnp.sum(vv3 * vv3, axis=1) + 1e-12)
    s = s_ref[...]
    h = (jnp.dot(s, w1s_ref[...], preferred_element_type=jnp.float32)
         + jnp.dot(vn, w1v_ref[...], preferred_element_type=jnp.float32)
         + b1_ref[...])
    h = h * jax.nn.sigmoid(h)
    a = jnp.dot(h, w2_ref[...], preferred_element_type=jnp.float32) + b2_ref[...]
    a_vv = a[:, :128]
    a_sv = a[:, 128:256]
    a_ss = a[:, 256:]
    so_ref[...] = s + a_sv * jnp.sum(uv3 * vv3, axis=1) + a_ss
    vo_ref[...] = (v.reshape(_NB, 3, 128)
                   + a_vv[:, None, :] * uv3).reshape(3 * _NB, 128)


def _upd_stage(s_mid, v_mid_flat, u, vw, w1s, w1v, b1, w2, b2):
    return pl.pallas_call(
        _upd_body,
        grid=(N // _NB,),
        in_specs=[
            pl.BlockSpec((_NB, 128), lambda i: (i, 0)),
            pl.BlockSpec((3 * _NB, 128), lambda i: (i, 0)),
            pl.BlockSpec((128, 128), lambda i: (0, 0)),
            pl.BlockSpec((128, 128), lambda i: (0, 0)),
            pl.BlockSpec((128, 128), lambda i: (0, 0)),
            pl.BlockSpec((128, 128), lambda i: (0, 0)),
            pl.BlockSpec((1, 128), lambda i: (0, 0)),
            pl.BlockSpec((128, 384), lambda i: (0, 0)),
            pl.BlockSpec((1, 384), lambda i: (0, 0)),
        ],
        out_specs=[
            pl.BlockSpec((_NB, 128), lambda i: (i, 0)),
            pl.BlockSpec((3 * _NB, 128), lambda i: (i, 0)),
        ],
        out_shape=[
            jax.ShapeDtypeStruct((N, 128), jnp.float32),
            jax.ShapeDtypeStruct((3 * N, 128), jnp.float32),
        ],
    )(s_mid, v_mid_flat, u, vw, w1s, w1v, b1[None, :], w2, b2[None, :])


def _ro_body(s_ref, w1_ref, b1_ref, w2_ref, b2_ref, o_ref):
    h = jnp.dot(s_ref[...], w1_ref[...], preferred_element_type=jnp.float32)
    h = h + b1_ref[...]
    h = h * jax.nn.sigmoid(h)
    ae = jnp.dot(h, w2_ref[...], preferred_element_type=jnp.float32) + b2_ref[...]
    em = jnp.sum(ae.reshape(NMOL, N // NMOL), axis=1)
    o_ref[...] = jnp.broadcast_to(
        jnp.concatenate([em, jnp.zeros((6,), jnp.float32)])[:, None], (16, 128))


def _readout(s, w1, b1, w2, b2):
    out = pl.pallas_call(
        _ro_body,
        grid=(1,),
        in_specs=[
            pl.BlockSpec((N, 128), lambda i: (0, 0)),
            pl.BlockSpec((128, 64), lambda i: (0, 0)),
            pl.BlockSpec((1, 64), lambda i: (0, 0)),
            pl.BlockSpec((64, 1), lambda i: (0, 0)),
            pl.BlockSpec((1, 1), lambda i: (0, 0)),
        ],
        out_specs=pl.BlockSpec((16, 128), lambda i: (0, 0)),
        out_shape=jax.ShapeDtypeStruct((16, 128), jnp.float32),
    )(s, w1, b1[None, :], w2, b2[None, :])
    return out[:NMOL, :1]


def _swish(x):
    return x * jax.nn.sigmoid(x)


_PERM = []
for _c in range(4):
    for _t in range(3):
        _PERM += list(range(128 * _t + 32 * _c, 128 * _t + 32 * _c + 32))


def kernel(nxyz, params, nbr_list):
    z = nxyz[:, 0].astype(jnp.int32)
    xyz = nxyz[:, 1:]
    src = nbr_list[:, 0].astype(jnp.int32)
    dst = nbr_list[:, 1].astype(jnp.int32)
    r_ij = xyz[dst] - xyz[src]
    d = jnp.sqrt(jnp.sum(r_ij ** 2, axis=-1) + 1e-12)
    unit = r_ij / d[:, None]
    unit4 = jnp.pad(unit, ((0, 0), (0, 1))).reshape(-1)
    offsets = jnp.linspace(0.0, CUTOFF, NG)
    coeff = -0.5 / (offsets[1] - offsets[0]) ** 2
    e_rbf = jnp.exp(coeff * (d[:, None] - offsets[None, :]) ** 2)
    f_cut = 0.5 * (jnp.cos(jnp.pi * d / CUTOFF) + 1.0) * (d < CUTOFF)
    s = params["embed"][z]
    v = jnp.zeros((N, 3, F), jnp.float32)
    zeros_acc = jnp.zeros((NPAD, 128), jnp.float32)
    pad64 = jnp.zeros((N, 64), jnp.float32)
    perm = jnp.asarray(_PERM)

    erbf21 = jnp.concatenate(
        [e_rbf * f_cut[:, None], f_cut[:, None],
         jnp.zeros((E, 3), jnp.float32)], axis=1)

    for lp in params["layers"]:
        phi = _phi_mlp(s, lp["phi_W1"], lp["phi_b1"],
                       lp["phi_W2"][:, perm], lp["phi_b2"][perm])
        wd21 = jnp.concatenate(
            [lp["dist_W"][:, perm], lp["dist_b"][perm][None, :],
             jnp.zeros((3, 384), jnp.float32)], axis=0)
        w_all = _w_matmul(erbf21, wd21)
        ds_parts = []
        dv_parts = []
        for c in range(4):
            phi_c = phi[:, 96 * c:96 * c + 96]
            w_c = w_all[:, 96 * c:96 * c + 96]
            v_c = v[:, :, 32 * c:32 * c + 32].reshape(N, 96)
            pv = jnp.concatenate([phi_c, v_c, pad64], axis=1)
            part = _edge_call(src, dst, pv, w_c, unit4, zeros_acc)
            tot = part[0, :N] + part[1, :N]
            ds_parts.append(tot[:, :32])
            dv_parts.append(tot[:, 32:].reshape(N, 3, 32))
        s_mid = s + jnp.concatenate(ds_parts, axis=1)
        v_mid = v + jnp.concatenate(dv_parts, axis=2)
        s, v_flat = _upd_stage(
            s_mid, v_mid.reshape(3 * N, 128), lp["U"], lp["V"],
            lp["upd_W1"][:128], lp["upd_W1"][128:], lp["upd_b1"],
            lp["upd_W2"], lp["upd_b2"])
        v = v_flat.reshape(N, 3, 128)

    energy = _readout(s, params["ro_W1"], params["ro_b1"],
                      params["ro_W2"], params["ro_b2"])
    return energy, s


# speedup vs baseline: 1.0731x; 1.0731x over previous
import jax
import jax.numpy as jnp
from jax import lax
from jax.experimental import pallas as pl
from jax.experimental.pallas import tpu as pltpu
from jax.experimental.pallas import tpu_sc as plsc

N = 10000
E = 320000
F = 128
NG = 20
CUTOFF = 5.0
NMOL = 10

NWORK = 32
CH = 64
CPW = 156
NSUB = 16
NPAD = 10112
RPS = NPAD // NSUB

_mesh = plsc.VectorSubcoreMesh(core_axis_name="c", subcore_axis_name="s",
                               num_cores=2, num_subcores=16)


def _edge_body(src_hbm, dst_hbm, phi_hbm, v_hbm, w_hbm, unit_hbm, zeros_hbm,
               out_hbm,
               is0, is1, id0, id1, ph0, ph1, vg0, vg1, w0, w1, u0, u1, msg,
               acc, sg0, sg1):
    cid = lax.axis_index("c")
    sid = lax.axis_index("s")
    wid = sid * 2 + cid
    pltpu.sync_copy(zeros_hbm.at[pl.ds(sid * RPS, RPS)],
                    acc.at[pl.ds(sid * RPS, RPS)])
    plsc.subcore_barrier()

    idx_s = (is0, is1)
    idx_d = (id0, id1)
    phr = (ph0, ph1)
    vgr = (vg0, vg1)
    wr = (w0, w1)
    ur = (u0, u1)
    sg = (sg0, sg1)
    start = wid * CPW

    def load(slot, chk):
        off = chk * CH
        pltpu.sync_copy(src_hbm.at[pl.ds(off, CH)], idx_s[slot])
        pltpu.sync_copy(dst_hbm.at[pl.ds(off, CH)], idx_d[slot])
        pltpu.async_copy(phi_hbm.at[idx_d[slot]], phr[slot], sg[slot])
        pltpu.async_copy(v_hbm.at[idx_d[slot]], vgr[slot], sg[slot])
        pltpu.sync_copy(w_hbm.at[pl.ds(off, CH), :], wr[slot])
        pltpu.sync_copy(unit_hbm.at[pl.ds(off * 4, CH * 4)], ur[slot])

    def wait_in(slot):
        pltpu.make_async_copy(phi_hbm.at[idx_d[slot]], phr[slot], sg[slot]).wait()
        pltpu.make_async_copy(v_hbm.at[idx_d[slot]], vgr[slot], sg[slot]).wait()

    def compute(slot):
        pvr = phr[slot]
        vvr = vgr[slot]
        wvr = wr[slot]
        uvr = ur[slot]

        def edge(e, c2):
            ph = [pvr[e, pl.ds(16 * t, 16)] for t in range(6)]
            wv = [wvr[e, pl.ds(16 * t, 16)] for t in range(6)]
            vv = [vvr[e, pl.ds(16 * t, 16)] for t in range(6)]
            msg[e, pl.ds(0, 16)] = ph[0] * wv[0]
            msg[e, pl.ds(16, 16)] = ph[1] * wv[1]
            avs = (ph[2] * wv[2], ph[3] * wv[3])
            avd = (ph[4] * wv[4], ph[5] * wv[5])
            uv = uvr[pl.ds(e * 4, 16)]
            for k in range(3):
                uk = jnp.full((16,), uv[k], jnp.float32)
                for h in range(2):
                    msg[e, pl.ds(32 + 32 * k + 16 * h, 16)] = (
                        avs[h] * vv[2 * k + h] + avd[h] * uk)
            return c2

        lax.fori_loop(0, CH, edge, 0)

    def scat(slot):
        pltpu.sync_copy(msg, acc.at[idx_s[slot]], add=True)

    load(0, start)

    def pair(j, carry):
        load(1, start + 2 * j + 1)
        wait_in(0)
        compute(0)
        scat(0)

        @pl.when(j < (CPW // 2 - 1))
        def _():
            load(0, start + 2 * j + 2)

        wait_in(1)
        compute(1)
        scat(1)
        return carry

    lax.fori_loop(0, CPW // 2, pair, 0)

    @pl.when(wid < 8)
    def _():
        load(0, NWORK * CPW + wid)
        wait_in(0)
        compute(0)
        scat(0)

    plsc.subcore_barrier()
    pltpu.sync_copy(acc.at[pl.ds(sid * RPS, RPS)],
                    out_hbm.at[cid, pl.ds(sid * RPS, RPS)])


_edge_call = pl.kernel(
    _edge_body,
    out_type=jax.ShapeDtypeStruct((2, NPAD, 128), jnp.float32),
    mesh=_mesh,
    scratch_types=[
        pltpu.VMEM((CH,), jnp.int32),
        pltpu.VMEM((CH,), jnp.int32),
        pltpu.VMEM((CH,), jnp.int32),
        pltpu.VMEM((CH,), jnp.int32),
        pltpu.VMEM((CH, 96), jnp.float32),
        pltpu.VMEM((CH, 96), jnp.float32),
        pltpu.VMEM((CH, 96), jnp.float32),
        pltpu.VMEM((CH, 96), jnp.float32),
        pltpu.VMEM((CH, 96), jnp.float32),
        pltpu.VMEM((CH, 96), jnp.float32),
        pltpu.VMEM((CH * 4,), jnp.float32),
        pltpu.VMEM((CH * 4,), jnp.float32),
        pltpu.VMEM((CH, 128), jnp.float32),
        pltpu.VMEM_SHARED((NPAD, 128), jnp.float32),
        pltpu.SemaphoreType.DMA,
        pltpu.SemaphoreType.DMA,
    ],
    compiler_params=pltpu.CompilerParams(use_tc_tiling_on_sc=False),
)


_NB = 1000
_EB = 2000


def _phi_body(s_ref, w1_ref, b1_ref, w2_ref, b2_ref, o_ref):
    h = jnp.dot(s_ref[...], w1_ref[...], preferred_element_type=jnp.float32)
    h = h + b1_ref[...]
    h = h * jax.nn.sigmoid(h)
    o_ref[...] = jnp.dot(h, w2_ref[...],
                         preferred_element_type=jnp.float32) + b2_ref[...]


def _phi_mlp(s, w1, b1, w2p, b2p):
    return pl.pallas_call(
        _phi_body,
        grid=(N // _NB,),
        in_specs=[
            pl.BlockSpec((_NB, 128), lambda i: (i, 0)),
            pl.BlockSpec((128, 128), lambda i: (0, 0)),
            pl.BlockSpec((1, 128), lambda i: (0, 0)),
            pl.BlockSpec((128, 384), lambda i: (0, 0)),
            pl.BlockSpec((1, 384), lambda i: (0, 0)),
        ],
        out_specs=pl.BlockSpec((_NB, 384), lambda i: (i, 0)),
        out_shape=jax.ShapeDtypeStruct((N, 384), jnp.float32),
    )(s, w1, b1[None, :], w2p, b2p[None, :])


def _w_body(erbf_ref, wd_ref, o_ref):
    o_ref[...] = jnp.dot(erbf_ref[...], wd_ref[...],
                         preferred_element_type=jnp.float32)


def _w_matmul(erbf21, wd21):
    return pl.pallas_call(
        _w_body,
        grid=(E // _EB,),
        in_specs=[
            pl.BlockSpec((_EB, 24), lambda i: (i, 0)),
            pl.BlockSpec((24, 384), lambda i: (0, 0)),
        ],
        out_specs=pl.BlockSpec((_EB, 384), lambda i: (i, 0)),
        out_shape=jax.ShapeDtypeStruct((E, 384), jnp.float32),
    )(erbf21, wd21)


def _upd_body(s_ref, v_ref, u_ref, vw_ref, w1s_ref, w1v_ref, b1_ref,
              w2_ref, b2_ref, so_ref, vo_ref):
    v = v_ref[...]
    uv = jnp.dot(v, u_ref[...], preferred_element_type=jnp.float32)
    vv = jnp.dot(v, vw_ref[...], preferred_element_type=jnp.float32)
    vv3 = vv.reshape(_NB, 3, 128)
    uv3 = uv.reshape(_NB, 3, 128)
    vn = jnp.sqrt(jnp.sum(vv3 * vv3, axis=1) + 1e-12)
    s = s_ref[...]
    h = (jnp.dot(s, w1s_ref[...], preferred_element_type=jnp.float32)
         + jnp.dot(vn, w1v_ref[...], preferred_element_type=jnp.float32)
         + b1_ref[...])
    h = h * jax.nn.sigmoid(h)
    a = jnp.dot(h, w2_ref[...], preferred_element_type=jnp.float32) + b2_ref[...]
    a_vv = a[:, :128]
    a_sv = a[:, 128:256]
    a_ss = a[:, 256:]
    so_ref[...] = s + a_sv * jnp.sum(uv3 * vv3, axis=1) + a_ss
    vo_ref[...] = (v.reshape(_NB, 3, 128)
                   + a_vv[:, None, :] * uv3).reshape(3 * _NB, 128)


def _upd_stage(s_mid, v_mid_flat, u, vw, w1s, w1v, b1, w2, b2):
    return pl.pallas_call(
        _upd_body,
        grid=(N // _NB,),
        in_specs=[
            pl.BlockSpec((_NB, 128), lambda i: (i, 0)),
            pl.BlockSpec((3 * _NB, 128), lambda i: (i, 0)),
            pl.BlockSpec((128, 128), lambda i: (0, 0)),
            pl.BlockSpec((128, 128), lambda i: (0, 0)),
            pl.BlockSpec((128, 128), lambda i: (0, 0)),
            pl.BlockSpec((128, 128), lambda i: (0, 0)),
            pl.BlockSpec((1, 128), lambda i: (0, 0)),
            pl.BlockSpec((128, 384), lambda i: (0, 0)),
            pl.BlockSpec((1, 384), lambda i: (0, 0)),
        ],
        out_specs=[
            pl.BlockSpec((_NB, 128), lambda i: (i, 0)),
            pl.BlockSpec((3 * _NB, 128), lambda i: (i, 0)),
        ],
        out_shape=[
            jax.ShapeDtypeStruct((N, 128), jnp.float32),
            jax.ShapeDtypeStruct((3 * N, 128), jnp.float32),
        ],
    )(s_mid, v_mid_flat, u, vw, w1s, w1v, b1[None, :], w2, b2[None, :])


def _ro_body(s_ref, w1_ref, b1_ref, w2_ref, b2_ref, o_ref):
    h = jnp.dot(s_ref[...], w1_ref[...], preferred_element_type=jnp.float32)
    h = h + b1_ref[...]
    h = h * jax.nn.sigmoid(h)
    ae = jnp.dot(h, w2_ref[...], preferred_element_type=jnp.float32) + b2_ref[...]
    em = jnp.sum(ae.reshape(NMOL, N // NMOL), axis=1)
    o_ref[...] = jnp.broadcast_to(
        jnp.concatenate([em, jnp.zeros((6,), jnp.float32)])[:, None], (16, 128))


def _readout(s, w1, b1, w2, b2):
    out = pl.pallas_call(
        _ro_body,
        grid=(1,),
        in_specs=[
            pl.BlockSpec((N, 128), lambda i: (0, 0)),
            pl.BlockSpec((128, 64), lambda i: (0, 0)),
            pl.BlockSpec((1, 64), lambda i: (0, 0)),
            pl.BlockSpec((64, 1), lambda i: (0, 0)),
            pl.BlockSpec((1, 1), lambda i: (0, 0)),
        ],
        out_specs=pl.BlockSpec((16, 128), lambda i: (0, 0)),
        out_shape=jax.ShapeDtypeStruct((16, 128), jnp.float32),
    )(s, w1, b1[None, :], w2, b2[None, :])
    return out[:NMOL, :1]


def _swish(x):
    return x * jax.nn.sigmoid(x)


_PERM = []
for _c in range(4):
    for _t in range(3):
        _PERM += list(range(128 * _t + 32 * _c, 128 * _t + 32 * _c + 32))


def kernel(nxyz, params, nbr_list):
    z = nxyz[:, 0].astype(jnp.int32)
    xyz = nxyz[:, 1:]
    src = nbr_list[:, 0].astype(jnp.int32)
    dst = nbr_list[:, 1].astype(jnp.int32)
    r_ij = xyz[dst] - xyz[src]
    d = jnp.sqrt(jnp.sum(r_ij ** 2, axis=-1) + 1e-12)
    unit = r_ij / d[:, None]
    unit4 = jnp.pad(unit, ((0, 0), (0, 1))).reshape(-1)
    offsets = jnp.linspace(0.0, CUTOFF, NG)
    coeff = -0.5 / (offsets[1] - offsets[0]) ** 2
    e_rbf = jnp.exp(coeff * (d[:, None] - offsets[None, :]) ** 2)
    f_cut = 0.5 * (jnp.cos(jnp.pi * d / CUTOFF) + 1.0) * (d < CUTOFF)
    s = params["embed"][z]
    v = jnp.zeros((N, 3, F), jnp.float32)
    zeros_acc = jnp.zeros((NPAD, 128), jnp.float32)
    perm = jnp.asarray(_PERM)

    erbf21 = jnp.concatenate(
        [e_rbf * f_cut[:, None], f_cut[:, None],
         jnp.zeros((E, 3), jnp.float32)], axis=1)

    for lp in params["layers"]:
        phi = _phi_mlp(s, lp["phi_W1"], lp["phi_b1"],
                       lp["phi_W2"][:, perm], lp["phi_b2"][perm])
        wd21 = jnp.concatenate(
            [lp["dist_W"][:, perm], lp["dist_b"][perm][None, :],
             jnp.zeros((3, 384), jnp.float32)], axis=0)
        w_all = _w_matmul(erbf21, wd21)
        ds_parts = []
        dv_parts = []
        for c in range(4):
            phi_c = phi[:, 96 * c:96 * c + 96]
            w_c = w_all[:, 96 * c:96 * c + 96]
            v_c = v[:, :, 32 * c:32 * c + 32].reshape(N, 96)
            part = _edge_call(src, dst, phi_c, v_c, w_c, unit4, zeros_acc)
            tot = part[0, :N] + part[1, :N]
            ds_parts.append(tot[:, :32])
            dv_parts.append(tot[:, 32:].reshape(N, 3, 32))
        s_mid = s + jnp.concatenate(ds_parts, axis=1)
        v_mid = v + jnp.concatenate(dv_parts, axis=2)
        s, v_flat = _upd_stage(
            s_mid, v_mid.reshape(3 * N, 128), lp["U"], lp["V"],
            lp["upd_W1"][:128], lp["upd_W1"][128:], lp["upd_b1"],
            lp["upd_W2"], lp["upd_b2"])
        v = v_flat.reshape(N, 3, 128)

    energy = _readout(s, params["ro_W1"], params["ro_b1"],
                      params["ro_W2"], params["ro_b2"])
    return energy, s


# speedup vs baseline: 1.1587x; 1.0797x over previous
import jax
import jax.numpy as jnp
from jax import lax
from jax.experimental import pallas as pl
from jax.experimental.pallas import tpu as pltpu
from jax.experimental.pallas import tpu_sc as plsc

N = 10000
E = 320000
F = 128
NG = 20
CUTOFF = 5.0
NMOL = 10

NWORK = 32
CH = 64
CPW = 156
NSUB = 16
NPAD = 10112
RPS = NPAD // NSUB

_mesh = plsc.VectorSubcoreMesh(core_axis_name="c", subcore_axis_name="s",
                               num_cores=2, num_subcores=16)


def _edge_body(src_hbm, dst_hbm, phi_hbm, v_hbm, w_hbm, unit_hbm, zeros_hbm,
               out_hbm,
               is0, is1, id0, id1, ph0, ph1, vg0, vg1, w0, w1, u0, u1, msg,
               acc, sg0, sg1):
    cid = lax.axis_index("c")
    sid = lax.axis_index("s")
    wid = sid * 2 + cid
    pltpu.sync_copy(zeros_hbm.at[pl.ds(sid * RPS, RPS)],
                    acc.at[pl.ds(sid * RPS, RPS)])
    plsc.subcore_barrier()

    idx_s = (is0, is1)
    idx_d = (id0, id1)
    phr = (ph0, ph1)
    vgr = (vg0, vg1)
    wr = (w0, w1)
    ur = (u0, u1)
    sg = (sg0, sg1)
    start = wid * CPW

    def load(slot, chk):
        off = chk * CH
        pltpu.sync_copy(src_hbm.at[pl.ds(off, CH)], idx_s[slot])
        pltpu.sync_copy(dst_hbm.at[pl.ds(off, CH)], idx_d[slot])
        pltpu.async_copy(phi_hbm.at[idx_d[slot]], phr[slot], sg[slot])
        pltpu.async_copy(v_hbm.at[idx_d[slot]], vgr[slot], sg[slot])
        pltpu.sync_copy(w_hbm.at[pl.ds(off, CH), :], wr[slot])
        pltpu.sync_copy(unit_hbm.at[pl.ds(off * 4, CH * 4)], ur[slot])

    def wait_in(slot):
        pltpu.make_async_copy(phi_hbm.at[idx_d[slot]], phr[slot], sg[slot]).wait()
        pltpu.make_async_copy(v_hbm.at[idx_d[slot]], vgr[slot], sg[slot]).wait()

    def compute(slot):
        pvr = phr[slot]
        vvr = vgr[slot]
        wvr = wr[slot]
        uvr = ur[slot]

        @plsc.parallel_loop(0, CH, 1, unroll=4)
        def edge(e):
            ph = [pvr[e, pl.ds(16 * t, 16)] for t in range(6)]
            wv = [wvr[e, pl.ds(16 * t, 16)] for t in range(6)]
            vv = [vvr[e, pl.ds(16 * t, 16)] for t in range(6)]
            msg[e, pl.ds(0, 16)] = ph[0] * wv[0]
            msg[e, pl.ds(16, 16)] = ph[1] * wv[1]
            avs = (ph[2] * wv[2], ph[3] * wv[3])
            avd = (ph[4] * wv[4], ph[5] * wv[5])
            uv = uvr[pl.ds(e * 4, 16)]
            for k in range(3):
                uk = jnp.full((16,), uv[k], jnp.float32)
                for h in range(2):
                    msg[e, pl.ds(32 + 32 * k + 16 * h, 16)] = (
                        avs[h] * vv[2 * k + h] + avd[h] * uk)

    def scat(slot):
        pltpu.sync_copy(msg, acc.at[idx_s[slot]], add=True)

    load(0, start)

    def pair(j, carry):
        load(1, start + 2 * j + 1)
        wait_in(0)
        compute(0)
        scat(0)

        @pl.when(j < (CPW // 2 - 1))
        def _():
            load(0, start + 2 * j + 2)

        wait_in(1)
        compute(1)
        scat(1)
        return carry

    lax.fori_loop(0, CPW // 2, pair, 0)

    @pl.when(wid < 8)
    def _():
        load(0, NWORK * CPW + wid)
        wait_in(0)
        compute(0)
        scat(0)

    plsc.subcore_barrier()
    pltpu.sync_copy(acc.at[pl.ds(sid * RPS, RPS)],
                    out_hbm.at[cid, pl.ds(sid * RPS, RPS)])


_edge_call = pl.kernel(
    _edge_body,
    out_type=jax.ShapeDtypeStruct((2, NPAD, 128), jnp.float32),
    mesh=_mesh,
    scratch_types=[
        pltpu.VMEM((CH,), jnp.int32),
        pltpu.VMEM((CH,), jnp.int32),
        pltpu.VMEM((CH,), jnp.int32),
        pltpu.VMEM((CH,), jnp.int32),
        pltpu.VMEM((CH, 96), jnp.float32),
        pltpu.VMEM((CH, 96), jnp.float32),
        pltpu.VMEM((CH, 96), jnp.float32),
        pltpu.VMEM((CH, 96), jnp.float32),
        pltpu.VMEM((CH, 96), jnp.float32),
        pltpu.VMEM((CH, 96), jnp.float32),
        pltpu.VMEM((CH * 4,), jnp.float32),
        pltpu.VMEM((CH * 4,), jnp.float32),
        pltpu.VMEM((CH, 128), jnp.float32),
        pltpu.VMEM_SHARED((NPAD, 128), jnp.float32),
        pltpu.SemaphoreType.DMA,
        pltpu.SemaphoreType.DMA,
    ],
    compiler_params=pltpu.CompilerParams(use_tc_tiling_on_sc=False),
)


_NB = 1000
_EB = 2000


def _phi_body(s_ref, w1_ref, b1_ref, w2_ref, b2_ref, o_ref):
    h = jnp.dot(s_ref[...], w1_ref[...], preferred_element_type=jnp.float32)
    h = h + b1_ref[...]
    h = h * jax.nn.sigmoid(h)
    o_ref[...] = jnp.dot(h, w2_ref[...],
                         preferred_element_type=jnp.float32) + b2_ref[...]


def _phi_mlp(s, w1, b1, w2p, b2p):
    return pl.pallas_call(
        _phi_body,
        grid=(N // _NB,),
        in_specs=[
            pl.BlockSpec((_NB, 128), lambda i: (i, 0)),
            pl.BlockSpec((128, 128), lambda i: (0, 0)),
            pl.BlockSpec((1, 128), lambda i: (0, 0)),
            pl.BlockSpec((128, 384), lambda i: (0, 0)),
            pl.BlockSpec((1, 384), lambda i: (0, 0)),
        ],
        out_specs=pl.BlockSpec((_NB, 384), lambda i: (i, 0)),
        out_shape=jax.ShapeDtypeStruct((N, 384), jnp.float32),
    )(s, w1, b1[None, :], w2p, b2p[None, :])


def _w_body(erbf_ref, wd_ref, o_ref):
    o_ref[...] = jnp.dot(erbf_ref[...], wd_ref[...],
                         preferred_element_type=jnp.float32)


def _w_matmul(erbf21, wd21):
    return pl.pallas_call(
        _w_body,
        grid=(E // _EB,),
        in_specs=[
            pl.BlockSpec((_EB, 24), lambda i: (i, 0)),
            pl.BlockSpec((24, 384), lambda i: (0, 0)),
        ],
        out_specs=pl.BlockSpec((_EB, 384), lambda i: (i, 0)),
        out_shape=jax.ShapeDtypeStruct((E, 384), jnp.float32),
    )(erbf21, wd21)


def _upd_body(s_ref, v_ref, u_ref, vw_ref, w1s_ref, w1v_ref, b1_ref,
              w2_ref, b2_ref, so_ref, vo_ref):
    v = v_ref[...]
    uv = jnp.dot(v, u_ref[...], preferred_element_type=jnp.float32)
    vv = jnp.dot(v, vw_ref[...], preferred_element_type=jnp.float32)
    vv3 = vv.reshape(_NB, 3, 128)
    uv3 = uv.reshape(_NB, 3, 128)
    vn = jnp.sqrt(jnp.sum(vv3 * vv3, axis=1) + 1e-12)
    s = s_ref[...]
    h = (jnp.dot(s, w1s_ref[...], preferred_element_type=jnp.float32)
         + jnp.dot(vn, w1v_ref[...], preferred_element_type=jnp.float32)
         + b1_ref[...])
    h = h * jax.nn.sigmoid(h)
    a = jnp.dot(h, w2_ref[...], preferred_element_type=jnp.float32) + b2_ref[...]
    a_vv = a[:, :128]
    a_sv = a[:, 128:256]
    a_ss = a[:, 256:]
    so_ref[...] = s + a_sv * jnp.sum(uv3 * vv3, axis=1) + a_ss
    vo_ref[...] = (v.reshape(_NB, 3, 128)
                   + a_vv[:, None, :] * uv3).reshape(3 * _NB, 128)


def _upd_stage(s_mid, v_mid_flat, u, vw, w1s, w1v, b1, w2, b2):
    return pl.pallas_call(
        _upd_body,
        grid=(N // _NB,),
        in_specs=[
            pl.BlockSpec((_NB, 128), lambda i: (i, 0)),
            pl.BlockSpec((3 * _NB, 128), lambda i: (i, 0)),
            pl.BlockSpec((128, 128), lambda i: (0, 0)),
            pl.BlockSpec((128, 128), lambda i: (0, 0)),
            pl.BlockSpec((128, 128), lambda i: (0, 0)),
            pl.BlockSpec((128, 128), lambda i: (0, 0)),
            pl.BlockSpec((1, 128), lambda i: (0, 0)),
            pl.BlockSpec((128, 384), lambda i: (0, 0)),
            pl.BlockSpec((1, 384), lambda i: (0, 0)),
        ],
        out_specs=[
            pl.BlockSpec((_NB, 128), lambda i: (i, 0)),
            pl.BlockSpec((3 * _NB, 128), lambda i: (i, 0)),
        ],
        out_shape=[
            jax.ShapeDtypeStruct((N, 128), jnp.float32),
            jax.ShapeDtypeStruct((3 * N, 128), jnp.float32),
        ],
    )(s_mid, v_mid_flat, u, vw, w1s, w1v, b1[None, :], w2, b2[None, :])


def _ro_body(s_ref, w1_ref, b1_ref, w2_ref, b2_ref, o_ref):
    h = jnp.dot(s_ref[...], w1_ref[...], preferred_element_type=jnp.float32)
    h = h + b1_ref[...]
    h = h * jax.nn.sigmoid(h)
    ae = jnp.dot(h, w2_ref[...], preferred_element_type=jnp.float32) + b2_ref[...]
    em = jnp.sum(ae.reshape(NMOL, N // NMOL), axis=1)
    o_ref[...] = jnp.broadcast_to(
        jnp.concatenate([em, jnp.zeros((6,), jnp.float32)])[:, None], (16, 128))


def _readout(s, w1, b1, w2, b2):
    out = pl.pallas_call(
        _ro_body,
        grid=(1,),
        in_specs=[
            pl.BlockSpec((N, 128), lambda i: (0, 0)),
            pl.BlockSpec((128, 64), lambda i: (0, 0)),
            pl.BlockSpec((1, 64), lambda i: (0, 0)),
            pl.BlockSpec((64, 1), lambda i: (0, 0)),
            pl.BlockSpec((1, 1), lambda i: (0, 0)),
        ],
        out_specs=pl.BlockSpec((16, 128), lambda i: (0, 0)),
        out_shape=jax.ShapeDtypeStruct((16, 128), jnp.float32),
    )(s, w1, b1[None, :], w2, b2[None, :])
    return out[:NMOL, :1]


def _swish(x):
    return x * jax.nn.sigmoid(x)


_PERM = []
for _c in range(4):
    for _t in range(3):
        _PERM += list(range(128 * _t + 32 * _c, 128 * _t + 32 * _c + 32))


def kernel(nxyz, params, nbr_list):
    z = nxyz[:, 0].astype(jnp.int32)
    xyz = nxyz[:, 1:]
    src = nbr_list[:, 0].astype(jnp.int32)
    dst = nbr_list[:, 1].astype(jnp.int32)
    r_ij = xyz[dst] - xyz[src]
    d = jnp.sqrt(jnp.sum(r_ij ** 2, axis=-1) + 1e-12)
    unit = r_ij / d[:, None]
    unit4 = jnp.pad(unit, ((0, 0), (0, 1))).reshape(-1)
    offsets = jnp.linspace(0.0, CUTOFF, NG)
    coeff = -0.5 / (offsets[1] - offsets[0]) ** 2
    e_rbf = jnp.exp(coeff * (d[:, None] - offsets[None, :]) ** 2)
    f_cut = 0.5 * (jnp.cos(jnp.pi * d / CUTOFF) + 1.0) * (d < CUTOFF)
    s = params["embed"][z]
    v = jnp.zeros((N, 3, F), jnp.float32)
    zeros_acc = jnp.zeros((NPAD, 128), jnp.float32)
    perm = jnp.asarray(_PERM)

    erbf21 = jnp.concatenate(
        [e_rbf * f_cut[:, None], f_cut[:, None],
         jnp.zeros((E, 3), jnp.float32)], axis=1)

    for lp in params["layers"]:
        phi = _phi_mlp(s, lp["phi_W1"], lp["phi_b1"],
                       lp["phi_W2"][:, perm], lp["phi_b2"][perm])
        wd21 = jnp.concatenate(
            [lp["dist_W"][:, perm], lp["dist_b"][perm][None, :],
             jnp.zeros((3, 384), jnp.float32)], axis=0)
        w_all = _w_matmul(erbf21, wd21)
        ds_parts = []
        dv_parts = []
        for c in range(4):
            phi_c = phi[:, 96 * c:96 * c + 96]
            w_c = w_all[:, 96 * c:96 * c + 96]
            v_c = v[:, :, 32 * c:32 * c + 32].reshape(N, 96)
            part = _edge_call(src, dst, phi_c, v_c, w_c, unit4, zeros_acc)
            tot = part[0, :N] + part[1, :N]
            ds_parts.append(tot[:, :32])
            dv_parts.append(tot[:, 32:].reshape(N, 3, 32))
        s_mid = s + jnp.concatenate(ds_parts, axis=1)
        v_mid = v + jnp.concatenate(dv_parts, axis=2)
        s, v_flat = _upd_stage(
            s_mid, v_mid.reshape(3 * N, 128), lp["U"], lp["V"],
            lp["upd_W1"][:128], lp["upd_W1"][128:], lp["upd_b1"],
            lp["upd_W2"], lp["upd_b2"])
        v = v_flat.reshape(N, 3, 128)

    energy = _readout(s, params["ro_W1"], params["ro_b1"],
                      params["ro_W2"], params["ro_b2"])
    return energy, s
